# baseline (speedup 1.0000x reference)
import functools

import jax
import jax.numpy as jnp
from jax import lax
from jax.experimental import pallas as pl
from jax.experimental.pallas import tpu as pltpu
from jax.experimental.pallas import tpu_sc as plsc

B, T, LND, CH = 64, 384, 543, 3
W = LND * CH
TAIL0 = 468 * CH
TAILW = 75 * CH
F = 150
FPAD = 160
OUTW = 3 * F
OPAD = 464
NW = 32
BPW = B // NW
TCH = 48
NCHUNK = T // TCH
NK = FPAD // 16
MEAN0 = 48


def _preprocess_sc(xt, xm):
    mesh = plsc.VectorSubcoreMesh(core_axis_name="c", subcore_axis_name="s")

    @functools.partial(
        pl.kernel,
        out_type=jax.ShapeDtypeStruct((B, T, OUTW), jnp.float32),
        mesh=mesh,
        compiler_params=pltpu.CompilerParams(use_tc_tiling_on_sc=False,
                                             needs_layout_passes=False),
        scratch_types=[
            pltpu.VMEM((T, 16), jnp.float32),
            pltpu.VMEM((TCH, TAILW), jnp.float32),
            pltpu.VMEM((T + 2, FPAD), jnp.float32),
            pltpu.VMEM((TCH, OUTW), jnp.float32),
        ],
    )
    def k(xt_hbm, xm_hbm, out_hbm, mbuf, rbuf, dbuf, obuf):
        cid = lax.axis_index("c")
        sid = lax.axis_index("s")
        wid = sid * 2 + cid

        lane = lax.iota(jnp.int32, 16)
        parity = (lane & 1) == 1
        colv = []
        wmask = []
        c1v = []
        c2v = []
        for kk in range(NK):
            f = lane + (16 * kk)
            col = 3 * (f >> 1) + (f & 1)
            colv.append(jnp.minimum(col, TAILW - 1))
            wmask.append(jnp.where(f < F, 1.0, 0.0).astype(jnp.float32))
            c1v.append(F + f)
            c2v.append(jnp.minimum(2 * F + f, OUTW - 1))
        tailmask = lane < (F - 16 * (NK - 1))
        zeros16 = jnp.zeros((16,), jnp.float32)
        m3 = jnp.where(lane == 3, 1.0, 0.0).astype(jnp.float32)
        m4 = jnp.where(lane == 4, 1.0, 0.0).astype(jnp.float32)
        even_f = jnp.where(parity, 0.0, 1.0).astype(jnp.float32)
        odd_f = jnp.where(parity, 1.0, 0.0).astype(jnp.float32)

        def batch_body(j, _):
            b = wid * BPW + j

            pltpu.sync_copy(xm_hbm.at[b], mbuf)

            def amean(r, acc):
                return acc + mbuf[r, :]

            acc = lax.fori_loop(0, T, amean, zeros16)
            m0 = jnp.sum(acc * m3) * (1.0 / T)
            m1 = jnp.sum(acc * m4) * (1.0 / T)
            mvec = jnp.where(parity, m1, m0)

            def bchunk(ci, accv):
                t0 = ci * TCH
                pltpu.sync_copy(xt_hbm.at[b, pl.ds(t0, TCH)], rbuf)

                def bbody(r, a):
                    rv = jnp.broadcast_to(r, (16,)).astype(jnp.int32)
                    for kk in range(NK):
                        xg = plsc.load_gather(rbuf, [rv, colv[kk]])
                        d = xg - mvec
                        dbuf[t0 + r, pl.ds(16 * kk, 16)] = d
                        a = a + d * d * wmask[kk]
                    return a

                return lax.fori_loop(0, TCH, bbody, accv)

            accv = lax.fori_loop(0, NCHUNK, bchunk, zeros16)

            s0 = jnp.sum(accv * even_f) * (1.0 / (T * 75))
            s1 = jnp.sum(accv * odd_f) * (1.0 / (T * 75))
            varv = jnp.where(parity, s1, s0)
            iv = plsc.bitcast(varv, jnp.int32)
            iv = jnp.full((16,), 0x5F3759DF, jnp.int32) - lax.shift_right_logical(iv, 1)
            y = plsc.bitcast(iv, jnp.float32)
            for _ in range(4):
                y = y * (1.5 - 0.5 * varv * y * y)
            svec = y

            for kk in range(NK):
                dbuf[T, pl.ds(16 * kk, 16)] = zeros16
                dbuf[T + 1, pl.ds(16 * kk, 16)] = zeros16

            def cchunk(ci, _):
                t0 = ci * TCH

                def cbody(r, carry):
                    t = t0 + r
                    rv = jnp.broadcast_to(r, (16,)).astype(jnp.int32)
                    avals = []
                    for kk in range(NK):
                        sl = pl.ds(16 * kk, 16)
                        a = dbuf[t, sl]
                        avals.append(a)
                        obuf[r, sl] = a * svec
                    for kk in range(NK):
                        b1 = dbuf[t + 1, pl.ds(16 * kk, 16)]
                        plsc.store_scatter(obuf, [rv, c1v[kk]],
                                           (b1 - avals[kk]) * svec)
                    for kk in range(NK):
                        b2 = dbuf[t + 2, pl.ds(16 * kk, 16)]
                        msk = tailmask if kk == NK - 1 else None
                        plsc.store_scatter(obuf, [rv, c2v[kk]],
                                           (b2 - avals[kk]) * svec, mask=msk)
                    return carry

                lax.fori_loop(0, TCH, cbody, 0)

                @pl.when(ci == NCHUNK - 1)
                def _zero_tail():
                    r1 = jnp.broadcast_to(TCH - 1, (16,)).astype(jnp.int32)
                    r2 = jnp.broadcast_to(TCH - 2, (16,)).astype(jnp.int32)
                    for kk in range(NK):
                        msk = tailmask if kk == NK - 1 else None
                        plsc.store_scatter(obuf, [r1, c1v[kk]], zeros16)
                        plsc.store_scatter(obuf, [r2, c2v[kk]], zeros16, mask=msk)
                        plsc.store_scatter(obuf, [r1, c2v[kk]], zeros16, mask=msk)

                pltpu.sync_copy(obuf, out_hbm.at[b, pl.ds(t0, TCH)])
                return 0

            lax.fori_loop(0, NCHUNK, cchunk, 0)
            return 0

        lax.fori_loop(0, BPW, batch_body, 0)

    return k(xt, xm)


def kernel(inputs):
    x3 = inputs.reshape(B, T, W)
    xt = lax.slice(x3, (0, 0, TAIL0), (B, T, TAIL0 + TAILW))
    xm = lax.slice(x3, (0, 0, MEAN0), (B, T, MEAN0 + 16))
    return _preprocess_sc(xt, xm)
